# baseline (speedup 1.0000x reference)
import functools

import jax
import jax.numpy as jnp
from jax import lax
from jax.experimental import pallas as pl
from jax.experimental.pallas import tpu as pltpu
from jax.experimental.pallas import tpu_sc as plsc

DIM = 64
BATCH = 16384
NC = 2
NS = 16
L = 16
NW = NC * NS
BPW = BATCH // NW
CHUNK = 128
NCHUNK = BPW // CHUNK

_mesh = plsc.VectorSubcoreMesh(core_axis_name="c", subcore_axis_name="s")


@functools.partial(
    pl.kernel,
    out_type=jax.ShapeDtypeStruct((BATCH,), jnp.float32),
    mesh=_mesh,
    compiler_params=pltpu.CompilerParams(
        needs_layout_passes=False, use_tc_tiling_on_sc=False),
    scratch_types=[
        pltpu.VMEM((NCHUNK, CHUNK), jnp.int32),
        pltpu.VMEM((NCHUNK, CHUNK), jnp.int32),
        pltpu.VMEM((BPW, DIM), jnp.float32),
        pltpu.VMEM((BPW, DIM), jnp.float32),
        pltpu.VMEM((BPW,), jnp.float32),
        pltpu.VMEM((BPW,), jnp.float32),
        pltpu.VMEM((BPW * L,), jnp.float32),
        pltpu.VMEM((BPW,), jnp.float32),
        pltpu.SemaphoreType.DMA,
    ],
)
def _glove_sc(i_idx2d, j_idx2d, wi, wj, bi_flat, bj_flat, out_hbm,
              idx_i, idx_j, rows_i, rows_j, bias_i, bias_j, partial,
              out_v, sem):
    wid = lax.axis_index("s") * NC + lax.axis_index("c")
    base = wid * BPW

    pltpu.sync_copy(i_idx2d.at[pl.ds(wid * NCHUNK, NCHUNK)], idx_i)
    pltpu.sync_copy(j_idx2d.at[pl.ds(wid * NCHUNK, NCHUNK)], idx_j)

    handles = []
    for k in range(NCHUNK):
        rows = pl.ds(k * CHUNK, CHUNK)
        handles.append(
            pltpu.async_copy(wi.at[idx_i.at[k]], rows_i.at[rows], sem))
        handles.append(
            pltpu.async_copy(wj.at[idx_j.at[k]], rows_j.at[rows], sem))
        handles.append(
            pltpu.async_copy(bi_flat.at[idx_i.at[k]], bias_i.at[rows], sem))
        handles.append(
            pltpu.async_copy(bj_flat.at[idx_j.at[k]], bias_j.at[rows], sem))
    for h in handles:
        h.wait()

    def body1(t, carry):
        acc = rows_i[t, pl.ds(0, L)] * rows_j[t, pl.ds(0, L)]
        for k in range(1, DIM // L):
            acc += rows_i[t, pl.ds(k * L, L)] * rows_j[t, pl.ds(k * L, L)]
        partial[pl.ds(pl.multiple_of(t * L, L), L)] = acc
        return carry

    lax.fori_loop(0, BPW, body1, 0)

    iota = lax.iota(jnp.int32, L)

    def body2(g, carry):
        acc = bias_i[pl.ds(g * L, L)] + bias_j[pl.ds(g * L, L)]
        flat_idx = g * (L * L) + iota * L
        for k in range(L):
            acc += plsc.load_gather(partial, [flat_idx + k])
        out_v[pl.ds(g * L, L)] = acc
        return carry

    lax.fori_loop(0, BPW // L, body2, 0)

    pltpu.sync_copy(out_v, out_hbm.at[pl.ds(base, BPW)])


def kernel(i_idx, j_idx, wi, wj, bi, bj):
    i2 = i_idx.astype(jnp.int32).reshape(NW * NCHUNK, CHUNK)
    j2 = j_idx.astype(jnp.int32).reshape(NW * NCHUNK, CHUNK)
    return _glove_sc(i2, j2, wi, wj, bi.reshape(-1), bj.reshape(-1))
